# baseline (speedup 1.0000x reference)
import math

import jax
import jax.numpy as jnp
from jax.experimental import pallas as pl
from jax.experimental.pallas import tpu as pltpu

_VOCAB = 101
_HIDDEN = 32
_NUM_HEADS = 2
_HEAD_DIM = _HIDDEN // _NUM_HEADS
_FFN = 64
_NUM_LAYERS = 2
_LN_EPS = 1e-12
_SEQ = 16
_VPAD = 128


def _mxu_dot(a, b):
    return jnp.dot(a.astype(jnp.bfloat16), b.astype(jnp.bfloat16),
                   preferred_element_type=jnp.float32)


def _layernorm(x, g, b, M):
    mu = jnp.dot(x, M, preferred_element_type=jnp.float32)
    m2 = jnp.dot(x * x, M, preferred_element_type=jnp.float32)
    var = jnp.maximum(m2 - mu * mu, 0.0)
    return (x - mu) * jax.lax.rsqrt(var + _LN_EPS) * g + b


def _fused_kernel(ids_ref, mask_pool_ref, mask_keys_ref,
                  wemb_hi_ref, wemb_lo_ref, posplus_ref, eg_ref, eb_ref,
                  wqkv_ref, bqkv_ref, wo_ref, bo_ref,
                  w1_ref, b1_ref, w2_ref, b2_ref,
                  ln1g_ref, ln1b_ref, ln2g_ref, ln2b_ref,
                  out_ref):
    R, S = ids_ref.shape
    T = R * S
    G = R // 8
    scale = 1.0 / math.sqrt(_HEAD_DIM)

    ids = ids_ref[...]
    hot = (ids[:, :, None] ==
           jax.lax.broadcasted_iota(jnp.int32, (R, S, _VPAD), 2))
    oh = jnp.where(hot, 1.0, 0.0).astype(jnp.bfloat16).reshape(T, _VPAD)
    emb = (jnp.dot(oh, wemb_hi_ref[...], preferred_element_type=jnp.float32)
           + jnp.dot(oh, wemb_lo_ref[...], preferred_element_type=jnp.float32))
    emb = (emb.reshape(R, S, _HIDDEN) + posplus_ref[...][None]).reshape(T, _HIDDEN)
    M = jnp.full((_HIDDEN, _HIDDEN), 1.0 / _HIDDEN, jnp.float32)
    ones_sum = jnp.full((_HIDDEN, _HIDDEN), 1.0, jnp.float32)
    ones_p = jnp.full((128, _HEAD_DIM), 1.0, jnp.float32)
    x = _layernorm(emb, eg_ref[...], eb_ref[...], M)

    qseq = jax.lax.broadcasted_iota(jnp.int32, (128, 128), 0) // _SEQ
    kseq = jax.lax.broadcasted_iota(jnp.int32, (128, 128), 1) // _SEQ
    same = (qseq == kseq)[None]
    mkf = mask_keys_ref[...].astype(jnp.float32)
    bias = jnp.where(same, (1.0 - mkf)[:, None, :] * (-1e9), -2e9)

    for l in range(_NUM_LAYERS):
        acc = jnp.zeros((T, _HIDDEN), jnp.float32)
        for h in range(_NUM_HEADS):
            q = _mxu_dot(x, wqkv_ref[l, h]) + bqkv_ref[l, h]
            k = _mxu_dot(x, wqkv_ref[l, _NUM_HEADS + h]) + bqkv_ref[l, _NUM_HEADS + h]
            v = _mxu_dot(x, wqkv_ref[l, 2 * _NUM_HEADS + h]) + bqkv_ref[l, 2 * _NUM_HEADS + h]
            qg = q.reshape(G, 128, _HEAD_DIM)
            kg = k.reshape(G, 128, _HEAD_DIM)
            vg = v.reshape(G, 128, _HEAD_DIM)
            s = jax.lax.dot_general(
                qg, kg, (((2,), (2,)), ((0,), (0,))),
                preferred_element_type=jnp.float32) * scale + bias
            p = jnp.exp(s)
            denom = jnp.dot(p.reshape(T, 128), ones_p,
                            preferred_element_type=jnp.float32)
            ctx = jax.lax.dot_general(
                p, vg, (((2,), (1,)), ((0,), (0,))),
                preferred_element_type=jnp.float32).reshape(T, _HEAD_DIM)
            ctx = ctx * pl.reciprocal(jnp.maximum(denom, 1e-30), approx=True)
            acc = acc + _mxu_dot(ctx, wo_ref[l, h])
        x1 = _layernorm(acc + bo_ref[l] + x, ln1g_ref[l], ln1b_ref[l], M)

        ff = _mxu_dot(x1, w1_ref[l]) + b1_ref[l]
        ff = jax.nn.gelu(ff, approximate=True)
        ff = _mxu_dot(ff, w2_ref[l]) + b2_ref[l]
        x = _layernorm(ff + x1, ln2g_ref[l], ln2b_ref[l], M)

    x3 = x.reshape(R, S, _HIDDEN)
    mpf = mask_pool_ref[...].astype(jnp.float32)
    summed = jnp.sum(x3 * mpf[:, :, None], axis=1)
    counts = jnp.dot(mpf, jnp.full((S, _HIDDEN), 1.0, jnp.float32),
                     preferred_element_type=jnp.float32)
    pooled = summed / jnp.maximum(counts, 1e-9)
    sq = jnp.dot(pooled * pooled, ones_sum,
                 preferred_element_type=jnp.float32)
    e = pooled * jax.lax.rsqrt(jnp.maximum(sq, 1e-24))

    e3 = e.reshape(G, 8, _HIDDEN)
    sc = jnp.sum(e3 * e3[:, 0:1, :], axis=-1)
    prob = 1.0 / (1.0 + jnp.exp(-sc))
    out_ref[...] = jnp.clip(prob, 1e-8, 1.0 - 1e-8)


def kernel(input_ids, att_mask, word_emb, pos_emb, type_emb, emb_ln_g, emb_ln_b,
           wqkv, bqkv, wo, bo, w1, b1, w2, b2, ln1_g, ln1_b, ln2_g, ln2_b):
    batch_size, num_samples, seq = input_ids.shape
    num_neg = num_samples - 2
    rows = batch_size * num_samples

    ids2 = input_ids.reshape(rows, seq)
    mask_pool = att_mask.reshape(rows, seq)
    mask_keys = att_mask.reshape(rows // 8, 8 * seq)

    wpad = jnp.zeros((_VPAD, _HIDDEN), jnp.float32).at[:_VOCAB].set(word_emb)
    wemb_hi = wpad.astype(jnp.bfloat16)
    wemb_lo = (wpad - wemb_hi.astype(jnp.float32)).astype(jnp.bfloat16)
    posplus = pos_emb[:seq] + type_emb[0][None, :]

    R = 256 if rows % 256 == 0 else 8
    grid = (rows // R,)
    G = R // 8

    def batched(shape2plus):
        nz = len(shape2plus) - 1
        return pl.BlockSpec(shape2plus, lambda i, nz=nz: (i,) + (0,) * nz)

    def full(arr):
        rank = arr.ndim
        return pl.BlockSpec(arr.shape, lambda i, rank=rank: (0,) * rank)

    consts = (wemb_hi, wemb_lo, posplus, emb_ln_g, emb_ln_b,
              wqkv, bqkv, wo, bo, w1, b1, w2, b2, ln1_g, ln1_b, ln2_g, ln2_b)

    out = pl.pallas_call(
        _fused_kernel,
        out_shape=jax.ShapeDtypeStruct((rows // 8, 8), jnp.float32),
        grid=grid,
        in_specs=([batched((R, seq)), batched((R, seq)), batched((G, 8 * seq))]
                  + [full(a) for a in consts]),
        out_specs=pl.BlockSpec((G, 8), lambda i: (i, 0)),
        compiler_params=pltpu.CompilerParams(
            dimension_semantics=("parallel",),
            vmem_limit_bytes=64 * 1024 * 1024),
    )(ids2, mask_pool, mask_keys, *consts)

    pos_out = out[:, 1]
    neg_out = out[:, 2:2 + num_neg]
    return pos_out, neg_out


# speedup vs baseline: 1.3467x; 1.3467x over previous
import math

import jax
import jax.numpy as jnp
from jax.experimental import pallas as pl
from jax.experimental.pallas import tpu as pltpu

_VOCAB = 101
_HIDDEN = 32
_NUM_HEADS = 2
_HEAD_DIM = _HIDDEN // _NUM_HEADS
_FFN = 64
_NUM_LAYERS = 2
_LN_EPS = 1e-12
_SEQ = 16
_VPAD = 128


def _dg(lhs, rhs, dims):
    return jax.lax.dot_general(lhs, rhs, dims,
                               preferred_element_type=jnp.float32)


def _bdot(a, b):
    return jax.lax.dot_general(a.astype(jnp.bfloat16), b.astype(jnp.bfloat16),
                               (((1,), (0,)), ((), ())),
                               preferred_element_type=jnp.float32)


def _dg_scores(k, q):
    return jax.lax.dot_general(k, q, (((0,), (0,)), ((1,), (1,))),
                               preferred_element_type=jnp.float32)


def _dg_ctx(v, p):
    return jax.lax.dot_general(v, p, (((2,), (1,)), ((1,), (0,))),
                               preferred_element_type=jnp.float32)


def _dg_wo(w, ctx):
    return jax.lax.dot_general(w.astype(jnp.bfloat16),
                               ctx.astype(jnp.bfloat16),
                               (((1,), (1,)), ((), ())),
                               preferred_element_type=jnp.float32)


def _dg_ln(M, x):
    return jax.lax.dot_general(M, x, (((1,), (0,)), ((), ())),
                               preferred_element_type=jnp.float32)


def _fused_kernel(ids_ref, mask_ref,
                  wembT_hi_ref, wembT_lo_ref, pospT_ref, egT_ref, ebT_ref,
                  wqkvT_ref, bqkvT_ref, woT_ref, boT_ref,
                  w1T_ref, b1T_ref, w2T_ref, b2T_ref,
                  ln1gT_ref, ln1bT_ref, ln2gT_ref, ln2bT_ref,
                  out_ref):
    G = ids_ref.shape[0]

    M32 = jnp.full((_HIDDEN, _HIDDEN), 1.0 / _HIDDEN, jnp.float32)

    def lnT(x, g_ref, b_ref, l=None):
        g = (g_ref[...] if l is None else g_ref[l])[:, None, :]
        b = (b_ref[...] if l is None else b_ref[l])[:, None, :]
        mu = _dg_ln(M32, x)
        m2 = _dg_ln(M32, x * x)
        var = jnp.maximum(m2 - mu * mu, 0.0)
        return (x - mu) * jax.lax.rsqrt(var + _LN_EPS) * g + b

    ids = ids_ref[...]
    hot = (jax.lax.broadcasted_iota(jnp.int32, (_VPAD, G, 128), 0)
           == ids[None])
    oh = jnp.where(hot, 1.0, 0.0).astype(jnp.bfloat16)
    emb = (_dg(wembT_hi_ref[...], oh, (((1,), (0,)), ((), ())))
           + _dg(wembT_lo_ref[...], oh, (((1,), (0,)), ((), ()))))
    emb = emb + pospT_ref[...][:, None, :]
    x = lnT(emb, egT_ref, ebT_ref)

    kseq = jax.lax.broadcasted_iota(jnp.int32, (128, 128), 0) // _SEQ
    qseq = jax.lax.broadcasted_iota(jnp.int32, (128, 128), 1) // _SEQ
    same = (kseq == qseq)[None]
    mkf = mask_ref[...].astype(jnp.float32)
    bias = jnp.where(same, (1.0 - mkf)[:, :, None] * (-1e9), -2e9)

    for l in range(_NUM_LAYERS):
        acc = boT_ref[l][:, None, :] + x
        for h in range(_NUM_HEADS):
            q = _bdot(wqkvT_ref[l, h], x) + bqkvT_ref[l, h][:, None, :]
            k = _bdot(wqkvT_ref[l, _NUM_HEADS + h], x) \
                + bqkvT_ref[l, _NUM_HEADS + h][:, None, :]
            v = _bdot(wqkvT_ref[l, 2 * _NUM_HEADS + h], x) \
                + bqkvT_ref[l, 2 * _NUM_HEADS + h][:, None, :]
            s = _dg_scores(k, q) + bias
            p = jnp.exp(s)
            v_aug = jnp.concatenate(
                [v, jnp.ones((1, G, 128), jnp.float32)], axis=0)
            ctx_aug = _dg_ctx(v_aug, p)
            rn = pl.reciprocal(
                jnp.maximum(ctx_aug[:, _HEAD_DIM:, :], 1e-30), approx=True)
            ctx = ctx_aug[:, :_HEAD_DIM, :] * rn
            acc = acc + _dg_wo(woT_ref[l, h], ctx)
        x1 = lnT(acc, ln1gT_ref, ln1bT_ref, l)

        ff = _bdot(w1T_ref[l], x1) + b1T_ref[l][:, None, :]
        ff = jax.nn.gelu(ff, approximate=True)
        ff = _bdot(w2T_ref[l], ff) + b2T_ref[l][:, None, :]
        x = lnT(ff + x1, ln2gT_ref, ln2bT_ref, l)

    seg = jnp.where(jax.lax.broadcasted_iota(jnp.int32, (128, 8), 0) // _SEQ
                    == jax.lax.broadcasted_iota(jnp.int32, (128, 8), 1),
                    1.0, 0.0)
    xm = x * mkf[None]
    summed = _dg(xm, seg, (((2,), (0,)), ((), ())))
    counts = _dg(mkf, seg, (((1,), (0,)), ((), ())))
    pooled = summed / jnp.maximum(counts, 1e-9)[None]
    ones_h = jnp.full((1, _HIDDEN), 1.0, jnp.float32)
    sq = _dg(ones_h, pooled * pooled, (((1,), (0,)), ((), ())))
    e = pooled * jax.lax.rsqrt(jnp.maximum(sq, 1e-24))
    sc = _dg(ones_h, e * e[:, :, 0:1], (((1,), (0,)), ((), ())))
    prob = 1.0 / (1.0 + jnp.exp(-sc.reshape(G, 8)))
    out_ref[...] = jnp.clip(prob, 1e-8, 1.0 - 1e-8)


def kernel(input_ids, att_mask, word_emb, pos_emb, type_emb, emb_ln_g, emb_ln_b,
           wqkv, bqkv, wo, bo, w1, b1, w2, b2, ln1_g, ln1_b, ln2_g, ln2_b):
    batch_size, num_samples, seq = input_ids.shape
    num_neg = num_samples - 2
    rows = batch_size * num_samples
    groups = rows // 8

    ids_l = input_ids.reshape(groups, 8 * seq)
    mask_l = att_mask.reshape(groups, 8 * seq)

    wpadT = jnp.zeros((_HIDDEN, _VPAD), jnp.float32).at[:, :_VOCAB].set(word_emb.T)
    wembT_hi = wpadT.astype(jnp.bfloat16)
    wembT_lo = (wpadT - wembT_hi.astype(jnp.float32)).astype(jnp.bfloat16)
    pospT = jnp.tile((pos_emb[:seq] + type_emb[0][None, :]).T, (1, 8))

    def bcast_h(a):
        return jnp.broadcast_to(jnp.swapaxes(a, -1, -2),
                                a.shape[:-2] + (a.shape[-1], 128))

    egT, ebT = bcast_h(emb_ln_g), bcast_h(emb_ln_b)
    scale = 1.0 / math.sqrt(_HEAD_DIM)
    qscale = jnp.concatenate(
        [jnp.full((1, _NUM_HEADS, 1, 1), scale, jnp.float32),
         jnp.ones((1, 2 * _NUM_HEADS, 1, 1), jnp.float32)], axis=1)
    wqkvT = jnp.swapaxes(wqkv * qscale, -1, -2)
    bqkvT = bcast_h(bqkv) * qscale
    woT = jnp.swapaxes(wo, -1, -2)
    boT = bcast_h(bo)
    w1T = jnp.swapaxes(w1, -1, -2)
    b1T = bcast_h(b1)
    w2T = jnp.swapaxes(w2, -1, -2)
    b2T = bcast_h(b2)
    ln1gT, ln1bT = bcast_h(ln1_g), bcast_h(ln1_b)
    ln2gT, ln2bT = bcast_h(ln2_g), bcast_h(ln2_b)

    if groups % 32 == 0:
        G = 32
    elif groups % 8 == 0:
        G = 8
    else:
        G = 1
    grid = (groups // G,)

    def batched(shape2plus):
        nz = len(shape2plus) - 1
        return pl.BlockSpec(shape2plus, lambda i, nz=nz: (i,) + (0,) * nz)

    def full(arr):
        rank = arr.ndim
        return pl.BlockSpec(arr.shape, lambda i, rank=rank: (0,) * rank)

    consts = (wembT_hi, wembT_lo, pospT, egT, ebT,
              wqkvT, bqkvT, woT, boT, w1T, b1T, w2T, b2T,
              ln1gT, ln1bT, ln2gT, ln2bT)

    out = pl.pallas_call(
        _fused_kernel,
        out_shape=jax.ShapeDtypeStruct((groups, 8), jnp.float32),
        grid=grid,
        in_specs=([batched((G, 8 * seq)), batched((G, 8 * seq))]
                  + [full(a) for a in consts]),
        out_specs=pl.BlockSpec((G, 8), lambda i: (i, 0)),
        compiler_params=pltpu.CompilerParams(
            dimension_semantics=("arbitrary",),
            fuse_transposed_lhs_in_matmul=True,
            vmem_limit_bytes=64 * 1024 * 1024),
    )(ids_l, mask_l, *consts)

    pos_out = out[:, 1]
    neg_out = out[:, 2:2 + num_neg]
    return pos_out, neg_out


# speedup vs baseline: 1.4517x; 1.0780x over previous
import math

import jax
import jax.numpy as jnp
from jax.experimental import pallas as pl
from jax.experimental.pallas import tpu as pltpu

_VOCAB = 101
_HIDDEN = 32
_NUM_HEADS = 2
_HEAD_DIM = _HIDDEN // _NUM_HEADS
_FFN = 64
_NUM_LAYERS = 2
_LN_EPS = 1e-12
_SEQ = 16
_VPAD = 128


def _dg(lhs, rhs, dims):
    return jax.lax.dot_general(lhs, rhs, dims,
                               preferred_element_type=jnp.float32)


def _bdot(a, b):
    return jax.lax.dot_general(a.astype(jnp.bfloat16), b.astype(jnp.bfloat16),
                               (((1,), (0,)), ((), ())),
                               preferred_element_type=jnp.float32)


def _dg_scores(k, q):
    return jax.lax.dot_general(k, q, (((0,), (0,)), ((1,), (1,))),
                               preferred_element_type=jnp.float32)


def _dg_ctx(v, p):
    return jax.lax.dot_general(v, p, (((2,), (1,)), ((1,), (0,))),
                               preferred_element_type=jnp.float32)


def _dg_wo(w, ctx):
    return jax.lax.dot_general(w.astype(jnp.bfloat16),
                               ctx.astype(jnp.bfloat16),
                               (((1,), (1,)), ((), ())),
                               preferred_element_type=jnp.float32)


def _dg_ln(M, x):
    return jax.lax.dot_general(M, x, (((1,), (0,)), ((), ())),
                               preferred_element_type=jnp.float32)


def _fused_kernel(ids_ref, mask_ref,
                  wembT_hi_ref, wembT_lo_ref, pospT_ref, egT_ref, ebT_ref,
                  wqkvT_ref, bqkvT_ref, woT_ref, boT_ref,
                  w1T_ref, b1T_ref, w2T_ref, b2T_ref,
                  ln1gT_ref, ln1bT_ref, ln2gT_ref, ln2bT_ref,
                  out_ref):
    G = ids_ref.shape[0]

    M32 = jnp.full((_HIDDEN, _HIDDEN), 1.0 / _HIDDEN, jnp.float32)

    def lnT(x, g_ref, b_ref, l=None):
        g = (g_ref[...] if l is None else g_ref[l])[:, None, :]
        b = (b_ref[...] if l is None else b_ref[l])[:, None, :]
        mu = _dg_ln(M32, x)
        m2 = _dg_ln(M32, x * x)
        var = jnp.maximum(m2 - mu * mu, 0.0)
        return (x - mu) * jax.lax.rsqrt(var + _LN_EPS) * g + b

    ids = ids_ref[...]
    hot = (jax.lax.broadcasted_iota(jnp.int32, (_VPAD, G, 128), 0)
           == ids[None])
    oh = jnp.where(hot, 1.0, 0.0).astype(jnp.bfloat16)
    emb = (_dg(wembT_hi_ref[...], oh, (((1,), (0,)), ((), ())))
           + _dg(wembT_lo_ref[...], oh, (((1,), (0,)), ((), ()))))
    emb = emb + pospT_ref[...][:, None, :]
    x = lnT(emb, egT_ref, ebT_ref)

    kseq = jax.lax.broadcasted_iota(jnp.int32, (128, 128), 0) // _SEQ
    qseq = jax.lax.broadcasted_iota(jnp.int32, (128, 128), 1) // _SEQ
    same = (kseq == qseq)[None]
    mkf = mask_ref[...].astype(jnp.float32)
    bias = jnp.where(same, (1.0 - mkf)[:, :, None] * (-1e9), -2e9)

    for l in range(_NUM_LAYERS):
        acc = boT_ref[l][:, None, :] + x
        for h in range(_NUM_HEADS):
            q = _bdot(wqkvT_ref[l, h], x) + bqkvT_ref[l, h][:, None, :]
            k = _bdot(wqkvT_ref[l, _NUM_HEADS + h], x) \
                + bqkvT_ref[l, _NUM_HEADS + h][:, None, :]
            v = _bdot(wqkvT_ref[l, 2 * _NUM_HEADS + h], x) \
                + bqkvT_ref[l, 2 * _NUM_HEADS + h][:, None, :]
            s = _dg_scores(k, q) + bias
            p = jnp.exp(s)
            v_aug = jnp.concatenate(
                [v, jnp.ones((1, G, 128), jnp.float32)], axis=0)
            ctx_aug = _dg_ctx(v_aug, p)
            rn = pl.reciprocal(
                jnp.maximum(ctx_aug[:, _HEAD_DIM:, :], 1e-30), approx=True)
            ctx = ctx_aug[:, :_HEAD_DIM, :] * rn
            acc = acc + _dg_wo(woT_ref[l, h], ctx)
        x1 = lnT(acc, ln1gT_ref, ln1bT_ref, l)

        ff = _bdot(w1T_ref[l], x1) + b1T_ref[l][:, None, :]
        ff = jax.nn.gelu(ff, approximate=True)
        ff = _bdot(w2T_ref[l], ff) + b2T_ref[l][:, None, :]
        x = lnT(ff + x1, ln2gT_ref, ln2bT_ref, l)

    seg = jnp.where(jax.lax.broadcasted_iota(jnp.int32, (128, 8), 0) // _SEQ
                    == jax.lax.broadcasted_iota(jnp.int32, (128, 8), 1),
                    1.0, 0.0)
    xm = x * mkf[None]
    summed = _dg(xm, seg, (((2,), (0,)), ((), ())))
    counts = _dg(mkf, seg, (((1,), (0,)), ((), ())))
    pooled = summed / jnp.maximum(counts, 1e-9)[None]
    ones_h = jnp.full((1, _HIDDEN), 1.0, jnp.float32)
    sq = _dg(ones_h, pooled * pooled, (((1,), (0,)), ((), ())))
    e = pooled * jax.lax.rsqrt(jnp.maximum(sq, 1e-24))
    sc = _dg(ones_h, e * e[:, :, 0:1], (((1,), (0,)), ((), ())))
    prob = 1.0 / (1.0 + jnp.exp(-sc.reshape(G, 8)))
    out_ref[...] = jnp.clip(prob, 1e-8, 1.0 - 1e-8)


def kernel(input_ids, att_mask, word_emb, pos_emb, type_emb, emb_ln_g, emb_ln_b,
           wqkv, bqkv, wo, bo, w1, b1, w2, b2, ln1_g, ln1_b, ln2_g, ln2_b):
    batch_size, num_samples, seq = input_ids.shape
    num_neg = num_samples - 2
    rows = batch_size * num_samples
    groups = rows // 8

    ids_l = input_ids.reshape(groups, 8 * seq)
    mask_l = att_mask.reshape(groups, 8 * seq)

    wpadT = jnp.zeros((_HIDDEN, _VPAD), jnp.float32).at[:, :_VOCAB].set(word_emb.T)
    wembT_hi = wpadT.astype(jnp.bfloat16)
    wembT_lo = (wpadT - wembT_hi.astype(jnp.float32)).astype(jnp.bfloat16)
    pospT = jnp.tile((pos_emb[:seq] + type_emb[0][None, :]).T, (1, 8))

    def bcast_h(a):
        return jnp.broadcast_to(jnp.swapaxes(a, -1, -2),
                                a.shape[:-2] + (a.shape[-1], 128))

    egT, ebT = bcast_h(emb_ln_g), bcast_h(emb_ln_b)
    scale = 1.0 / math.sqrt(_HEAD_DIM)
    qscale = jnp.concatenate(
        [jnp.full((1, _NUM_HEADS, 1, 1), scale, jnp.float32),
         jnp.ones((1, 2 * _NUM_HEADS, 1, 1), jnp.float32)], axis=1)
    wqkvT = jnp.swapaxes(wqkv * qscale, -1, -2)
    bqkvT = bcast_h(bqkv) * qscale
    woT = jnp.swapaxes(wo, -1, -2)
    boT = bcast_h(bo)
    w1T = jnp.swapaxes(w1, -1, -2)
    b1T = bcast_h(b1)
    w2T = jnp.swapaxes(w2, -1, -2)
    b2T = bcast_h(b2)
    ln1gT, ln1bT = bcast_h(ln1_g), bcast_h(ln1_b)
    ln2gT, ln2bT = bcast_h(ln2_g), bcast_h(ln2_b)

    if groups % 64 == 0:
        G = 64
    elif groups % 32 == 0:
        G = 32
    elif groups % 8 == 0:
        G = 8
    else:
        G = 1
    grid = (groups // G,)

    def batched(shape2plus):
        nz = len(shape2plus) - 1
        return pl.BlockSpec(shape2plus, lambda i, nz=nz: (i,) + (0,) * nz)

    def full(arr):
        rank = arr.ndim
        return pl.BlockSpec(arr.shape, lambda i, rank=rank: (0,) * rank)

    consts = (wembT_hi, wembT_lo, pospT, egT, ebT,
              wqkvT, bqkvT, woT, boT, w1T, b1T, w2T, b2T,
              ln1gT, ln1bT, ln2gT, ln2bT)

    out = pl.pallas_call(
        _fused_kernel,
        out_shape=jax.ShapeDtypeStruct((groups, 8), jnp.float32),
        grid=grid,
        in_specs=([batched((G, 8 * seq)), batched((G, 8 * seq))]
                  + [full(a) for a in consts]),
        out_specs=pl.BlockSpec((G, 8), lambda i: (i, 0)),
        compiler_params=pltpu.CompilerParams(
            dimension_semantics=("arbitrary",),
            fuse_transposed_lhs_in_matmul=True,
            vmem_limit_bytes=64 * 1024 * 1024),
    )(ids_l, mask_l, *consts)

    pos_out = out[:, 1]
    neg_out = out[:, 2:2 + num_neg]
    return pos_out, neg_out
